# baseline (speedup 1.0000x reference)
import jax
import jax.numpy as jnp
from jax.experimental import pallas as pl
from jax.experimental.pallas import tpu as pltpu

_IN_F, _H1_F, _H2_F = 245, 120, 84


def _mlp_body(xt_ref, w1_ref, b1_ref, w2_ref, b2_ref, w3_ref, b3_ref, o_ref):
    xb = xt_ref[...].astype(jnp.bfloat16)
    h1 = jnp.tanh(
        jnp.dot(w1_ref[...].astype(jnp.bfloat16), xb,
                preferred_element_type=jnp.float32)
        + b1_ref[...]
    )
    h2 = jnp.tanh(
        jnp.dot(w2_ref[...].astype(jnp.bfloat16), h1.astype(jnp.bfloat16),
                preferred_element_type=jnp.float32)
        + b2_ref[...]
    )
    y = jnp.maximum(
        jnp.dot(w3_ref[...].astype(jnp.bfloat16), h2.astype(jnp.bfloat16),
                preferred_element_type=jnp.float32)
        + b3_ref[...],
        0.0,
    )
    o_ref[...] = y.reshape(o_ref.shape).astype(o_ref.dtype)


def _round_up(n, m):
    return ((n + m - 1) // m) * m


def kernel(x, w1, b1, w2, b2, w3, b3, *, tb=4096):
    B = x.shape[0]

    xt = x.T
    w1t = w1.T
    b1t = b1.T
    w2t = w2.T
    b2t = b2.T
    w3t = w3.T

    TB = min(tb, _round_up(B, 128))
    B_pad = _round_up(B, TB)
    xtp = xt if B_pad == B else jnp.pad(xt, ((0, 0), (0, B_pad - B)))

    out = pl.pallas_call(
        _mlp_body,
        out_shape=jax.ShapeDtypeStruct((B_pad // 128, 128), jnp.float32),
        grid=(B_pad // TB,),
        in_specs=[
            pl.BlockSpec((_IN_F, TB), lambda i: (0, i)),
            pl.BlockSpec((_H1_F, _IN_F), lambda i: (0, 0)),
            pl.BlockSpec((_H1_F, 1), lambda i: (0, 0)),
            pl.BlockSpec((_H2_F, _H1_F), lambda i: (0, 0)),
            pl.BlockSpec((_H2_F, 1), lambda i: (0, 0)),
            pl.BlockSpec((1, _H2_F), lambda i: (0, 0)),
            pl.BlockSpec((1, 1), lambda i: (0, 0)),
        ],
        out_specs=pl.BlockSpec((TB // 128, 128), lambda i: (i, 0)),
        compiler_params=pltpu.CompilerParams(
            dimension_semantics=("arbitrary",)
        ),
    )(xtp, w1t, b1t, w2t, b2t, w3t, b3)

    return out.reshape(B_pad, 1)[:B]


# speedup vs baseline: 1.1301x; 1.1301x over previous
import jax
import jax.numpy as jnp
from jax.experimental import pallas as pl
from jax.experimental.pallas import tpu as pltpu

_IN_F, _H1_F, _H2_F = 245, 120, 84


def _mlp_body(xt_ref, w1_ref, b1_ref, w2_ref, b2_ref, w3_ref, b3_ref, o_ref):
    xb = xt_ref[...].astype(jnp.bfloat16)
    h1 = jnp.tanh(
        jnp.dot(w1_ref[...].astype(jnp.bfloat16), xb,
                preferred_element_type=jnp.float32)
        + b1_ref[...]
    )
    h2 = jnp.tanh(
        jnp.dot(w2_ref[...].astype(jnp.bfloat16), h1.astype(jnp.bfloat16),
                preferred_element_type=jnp.float32)
        + b2_ref[...]
    )
    y = jnp.maximum(
        jnp.dot(w3_ref[...].astype(jnp.bfloat16), h2.astype(jnp.bfloat16),
                preferred_element_type=jnp.float32)
        + b3_ref[...],
        0.0,
    )
    o_ref[...] = y.reshape(o_ref.shape).astype(o_ref.dtype)


def _round_up(n, m):
    return ((n + m - 1) // m) * m


def kernel(x, w1, b1, w2, b2, w3, b3, *, tb=8192):
    B = x.shape[0]

    xt = x.T
    w1t = w1.T
    b1t = b1.T
    w2t = w2.T
    b2t = b2.T
    w3t = w3.T

    TB = min(tb, _round_up(B, 128))
    B_pad = _round_up(B, TB)
    xtp = xt if B_pad == B else jnp.pad(xt, ((0, 0), (0, B_pad - B)))

    out = pl.pallas_call(
        _mlp_body,
        out_shape=jax.ShapeDtypeStruct((B_pad // 128, 128), jnp.float32),
        grid=(B_pad // TB,),
        in_specs=[
            pl.BlockSpec((_IN_F, TB), lambda i: (0, i)),
            pl.BlockSpec((_H1_F, _IN_F), lambda i: (0, 0)),
            pl.BlockSpec((_H1_F, 1), lambda i: (0, 0)),
            pl.BlockSpec((_H2_F, _H1_F), lambda i: (0, 0)),
            pl.BlockSpec((_H2_F, 1), lambda i: (0, 0)),
            pl.BlockSpec((1, _H2_F), lambda i: (0, 0)),
            pl.BlockSpec((1, 1), lambda i: (0, 0)),
        ],
        out_specs=pl.BlockSpec((TB // 128, 128), lambda i: (i, 0)),
        compiler_params=pltpu.CompilerParams(
            dimension_semantics=("arbitrary",)
        ),
    )(xtp, w1t, b1t, w2t, b2t, w3t, b3)

    return out.reshape(B_pad, 1)[:B]
